# baseline (speedup 1.0000x reference)
import functools

import jax
import jax.numpy as jnp
from jax import lax
from jax.experimental import pallas as pl
from jax.experimental.pallas import tpu as pltpu
from jax.experimental.pallas import tpu_sc as plsc

_NC = 2
_NS = 16
_NW = _NC * _NS
_CH = 128
_GG = 4
_L = 16

_BN = 1000
_BE = 2048


def _silu(v):
    return v * jax.nn.sigmoid(v)


def _tc_node_init(x, cp, w_in, b_in, w1a, w1b):
    n, d_in = x.shape
    h = w_in.shape[1]
    grid = n // _BN

    def body(x_ref, cp_ref, win_ref, bin_ref, w1a_ref, w1b_ref,
             h_ref, t1_ref, t2_ref):
        h0 = jnp.dot(x_ref[...], win_ref[...],
                     preferred_element_type=jnp.float32) + bin_ref[...]
        h_ref[...] = h0
        cpv = cp_ref[...]
        p1 = jnp.dot(h0, w1a_ref[...], preferred_element_type=jnp.float32)
        p2 = jnp.dot(h0, w1b_ref[...], preferred_element_type=jnp.float32)
        t1_ref[...] = jnp.concatenate([p1, cpv], axis=1)
        t2_ref[...] = jnp.concatenate([p2, -cpv], axis=1)

    return pl.pallas_call(
        body,
        grid=(grid,),
        in_specs=[
            pl.BlockSpec((_BN, d_in), lambda i: (i, 0)),
            pl.BlockSpec((_BN, 16), lambda i: (i, 0)),
            pl.BlockSpec((d_in, h), lambda i: (0, 0)),
            pl.BlockSpec((1, h), lambda i: (0, 0)),
            pl.BlockSpec((h, h), lambda i: (0, 0)),
            pl.BlockSpec((h, h), lambda i: (0, 0)),
        ],
        out_specs=[
            pl.BlockSpec((_BN, h), lambda i: (i, 0)),
            pl.BlockSpec((_BN, 80), lambda i: (i, 0)),
            pl.BlockSpec((_BN, 80), lambda i: (i, 0)),
        ],
        out_shape=[
            jax.ShapeDtypeStruct((n, h), jnp.float32),
            jax.ShapeDtypeStruct((n, 80), jnp.float32),
            jax.ShapeDtypeStruct((n, 80), jnp.float32),
        ],
    )(x, cp, w_in, b_in, w1a, w1b)


def _tc_edge_mlp(gs, ea, wr, w3, be1, we2, be2, wc1, bc1, wc2):
    e = gs.shape[0]
    h = 64
    d_edge = ea.shape[1]
    grid = (e + _BE - 1) // _BE

    def body(gs_ref, ea_ref, wr_ref, w3_ref, be1_ref,
             we2_ref, be2_ref, wc1_ref, bc1_ref, wc2_ref, s_ref):
        gsv = gs_ref[...]
        g12 = gsv[:, 0:64]
        diff = gsv[:, 64:80]
        lane = lax.broadcasted_iota(jnp.int32, (_BE, 16), 1)
        mask3 = (lane < 3).astype(jnp.float32)
        diffm = diff * mask3
        radial = jnp.sum(diffm * diffm, axis=1, keepdims=True)
        pre = (g12 + radial * wr_ref[...]
               + jnp.dot(ea_ref[...], w3_ref[...],
                         preferred_element_type=jnp.float32) + be1_ref[...])
        m1 = _silu(pre)
        m = _silu(jnp.dot(m1, we2_ref[...],
                          preferred_element_type=jnp.float32) + be2_ref[...])
        t1 = _silu(jnp.dot(m, wc1_ref[...],
                           preferred_element_type=jnp.float32) + bc1_ref[...])
        tt = jnp.dot(t1, wc2_ref[...], preferred_element_type=jnp.float32)
        right = diffm * tt + (lane == 3).astype(jnp.float32)
        z = jnp.zeros((_BE, 48), jnp.float32)
        s_ref[...] = jnp.concatenate([m, right, z], axis=1)

    return pl.pallas_call(
        body,
        grid=(grid,),
        in_specs=[
            pl.BlockSpec((_BE, 128), lambda i: (i, 0)),
            pl.BlockSpec((_BE, d_edge), lambda i: (i, 0)),
            pl.BlockSpec((1, h), lambda i: (0, 0)),
            pl.BlockSpec((d_edge, h), lambda i: (0, 0)),
            pl.BlockSpec((1, h), lambda i: (0, 0)),
            pl.BlockSpec((h, h), lambda i: (0, 0)),
            pl.BlockSpec((1, h), lambda i: (0, 0)),
            pl.BlockSpec((h, h), lambda i: (0, 0)),
            pl.BlockSpec((1, h), lambda i: (0, 0)),
            pl.BlockSpec((h, 1), lambda i: (0, 0)),
        ],
        out_specs=pl.BlockSpec((_BE, 128), lambda i: (i, 0)),
        out_shape=jax.ShapeDtypeStruct((e, 128), jnp.float32),
    )(gs, ea, wr, w3, be1, we2, be2, wc1, bc1, wc2)


def _tc_node_update(hcur, t1_prev, acc2, wn1a, wn1b, bn1, wn2, bn2, w1a, w1b):
    n, h = hcur.shape

    def body(h_ref, t1p_ref, a_ref, wn1a_ref, wn1b_ref, bn1_ref, wn2_ref,
             bn2_ref, w1a_ref, w1b_ref, ho_ref, t1_ref, t2_ref):
        acc = a_ref[0] + a_ref[1]
        m_agg = acc[:, 0:64]
        right = acc[:, 64:80]
        cnt = right[:, 3:4]
        lane = lax.broadcasted_iota(jnp.int32, (_BN, 16), 1)
        mask3 = (lane < 3).astype(jnp.float32)
        num = right * mask3
        cpn = t1p_ref[:, 64:80] + num / jnp.maximum(cnt, 1.0)
        pre = (jnp.dot(h_ref[...], wn1a_ref[...], preferred_element_type=jnp.float32)
               + jnp.dot(m_agg, wn1b_ref[...], preferred_element_type=jnp.float32)
               + bn1_ref[...])
        hn = jnp.dot(_silu(pre), wn2_ref[...],
                     preferred_element_type=jnp.float32) + bn2_ref[...]
        hnew = h_ref[...] + hn
        ho_ref[...] = hnew
        p1 = jnp.dot(hnew, w1a_ref[...], preferred_element_type=jnp.float32)
        p2 = jnp.dot(hnew, w1b_ref[...], preferred_element_type=jnp.float32)
        t1_ref[...] = jnp.concatenate([p1, cpn], axis=1)
        t2_ref[...] = jnp.concatenate([p2, -cpn], axis=1)

    grid = n // _BN
    return pl.pallas_call(
        body,
        grid=(grid,),
        in_specs=[
            pl.BlockSpec((_BN, h), lambda i: (i, 0)),
            pl.BlockSpec((_BN, 80), lambda i: (i, 0)),
            pl.BlockSpec((2, _BN, 80), lambda i: (0, i, 0)),
            pl.BlockSpec((h, h), lambda i: (0, 0)),
            pl.BlockSpec((h, h), lambda i: (0, 0)),
            pl.BlockSpec((1, h), lambda i: (0, 0)),
            pl.BlockSpec((h, h), lambda i: (0, 0)),
            pl.BlockSpec((1, h), lambda i: (0, 0)),
            pl.BlockSpec((h, h), lambda i: (0, 0)),
            pl.BlockSpec((h, h), lambda i: (0, 0)),
        ],
        out_specs=[
            pl.BlockSpec((_BN, h), lambda i: (i, 0)),
            pl.BlockSpec((_BN, 80), lambda i: (i, 0)),
            pl.BlockSpec((_BN, 80), lambda i: (i, 0)),
        ],
        out_shape=[
            jax.ShapeDtypeStruct((n, h), jnp.float32),
            jax.ShapeDtypeStruct((n, 80), jnp.float32),
            jax.ShapeDtypeStruct((n, 80), jnp.float32),
        ],
    )(hcur, t1_prev, acc2, wn1a, wn1b, bn1, wn2, bn2, w1a, w1b)


def _tc_final(hcur, w_out, b_out):
    n, h = hcur.shape
    d_out = w_out.shape[1]
    grid = n // _BN

    def body(h_ref, w_ref, b_ref, o_ref):
        o_ref[...] = jnp.dot(h_ref[...], w_ref[...],
                             preferred_element_type=jnp.float32) + b_ref[...]

    return pl.pallas_call(
        body,
        grid=(grid,),
        in_specs=[
            pl.BlockSpec((_BN, h), lambda i: (i, 0)),
            pl.BlockSpec((h, d_out), lambda i: (0, 0)),
            pl.BlockSpec((1, d_out), lambda i: (0, 0)),
        ],
        out_specs=pl.BlockSpec((_BN, d_out), lambda i: (i, 0)),
        out_shape=jax.ShapeDtypeStruct((n, d_out), jnp.float32),
    )(hcur, w_out, b_out)


def _sc_gather(t1, t2, grow, gcol):
    e = grow.shape[0]
    step = _GG * _CH
    epw = e // _NW
    nj = epw // step
    rem = epw - nj * step
    mesh = plsc.VectorSubcoreMesh(core_axis_name="c", subcore_axis_name="s",
                                  num_cores=_NC, num_subcores=_NS)

    @functools.partial(
        pl.kernel,
        out_type=jax.ShapeDtypeStruct((e, 128), jnp.float32),
        mesh=mesh,
        compiler_params=pltpu.CompilerParams(use_tc_tiling_on_sc=False),
        scratch_types=[
            pltpu.VMEM((2, _GG, _CH), jnp.int32),
            pltpu.VMEM((2, _GG, _CH), jnp.int32),
            pltpu.VMEM((2, step, 80), jnp.float32),
            pltpu.SemaphoreType.DMA,
            pltpu.SemaphoreType.DMA,
            pltpu.SemaphoreType.DMA,
            pltpu.SemaphoreType.DMA,
            pltpu.SemaphoreType.DMA,
            pltpu.SemaphoreType.DMA,
        ],
    )
    def k(t1_h, t2_h, grow_h, gcol_h, gs_h,
          ir_v, ic_v, dr_v, sg0, sg1, sa0, sa1, sw0, sw1):
        c = lax.axis_index("c")
        s = lax.axis_index("s")
        wid = s * _NC + c
        base = wid * epw
        sgs = (sg0, sg1)
        sas = (sa0, sa1)
        sws = (sw0, sw1)

        def load_idx(slot, j):
            for g in range(_GG):
                off = base + j * step + g * _CH
                pltpu.sync_copy(grow_h.at[pl.ds(off, _CH)], ir_v.at[slot, g])
                pltpu.sync_copy(gcol_h.at[pl.ds(off, _CH)], ic_v.at[slot, g])

        def fire1(slot):
            for g in range(_GG):
                pltpu.async_copy(t1_h.at[ir_v.at[slot, g]],
                                 dr_v.at[slot, pl.ds(g * _CH, _CH)], sgs[slot])

        def wait1(slot):
            pltpu.make_async_copy(t1_h.at[ir_v.at[slot, 0]],
                                  dr_v.at[slot], sgs[slot]).wait()

        def fire2(slot):
            for g in range(_GG):
                pltpu.async_copy(t2_h.at[ic_v.at[slot, g]],
                                 dr_v.at[slot, pl.ds(g * _CH, _CH)], sas[slot],
                                 add=True)

        def wait2(slot):
            pltpu.make_async_copy(t2_h.at[ic_v.at[slot, 0]],
                                  dr_v.at[slot], sas[slot]).wait()

        def write(slot, j):
            off = base + j * step
            pltpu.async_copy(dr_v.at[slot],
                             gs_h.at[pl.ds(off, step), pl.ds(0, 80)], sws[slot])

        def wait_w(slot):
            pltpu.make_async_copy(dr_v.at[slot],
                                  gs_h.at[pl.ds(base, step), pl.ds(0, 80)],
                                  sws[slot]).wait()

        load_idx(0, 0)
        fire1(0)

        def body(jj, carry):
            a = jj * 2
            b = a + 1

            @pl.when(b < nj)
            def _():
                @pl.when(jj > 0)
                def _():
                    wait_w(1)

                load_idx(1, b)
                fire1(1)

            wait1(0)
            fire2(0)
            wait2(0)
            write(0, a)

            @pl.when(a + 2 < nj)
            def _():
                wait_w(0)
                load_idx(0, a + 2)
                fire1(0)

            @pl.when(b < nj)
            def _():
                wait1(1)
                fire2(1)
                wait2(1)
                write(1, b)

            return carry

        lax.fori_loop(0, (nj + 1) // 2, body, 0)
        wait_w(0)

        @pl.when(nj > 1)
        def _():
            wait_w(1)

        roff = base + nj * step
        nrc = rem // _CH
        tail = rem - nrc * _CH
        for q in range(nrc):
            off = roff + q * _CH
            pltpu.sync_copy(grow_h.at[pl.ds(off, _CH)], ir_v.at[0, 0])
            pltpu.sync_copy(gcol_h.at[pl.ds(off, _CH)], ic_v.at[0, 0])
            pltpu.async_copy(t1_h.at[ir_v.at[0, 0]],
                             dr_v.at[0, pl.ds(0, _CH)], sg0).wait()
            pltpu.async_copy(t2_h.at[ic_v.at[0, 0]],
                             dr_v.at[0, pl.ds(0, _CH)], sa0, add=True).wait()
            pltpu.sync_copy(dr_v.at[0, pl.ds(0, _CH)],
                            gs_h.at[pl.ds(off, _CH), pl.ds(0, 80)])
        if tail:
            off = roff + nrc * _CH
            pltpu.sync_copy(grow_h.at[pl.ds(off, tail)],
                            ir_v.at[0, 0, pl.ds(0, tail)])
            pltpu.sync_copy(gcol_h.at[pl.ds(off, tail)],
                            ic_v.at[0, 0, pl.ds(0, tail)])
            pltpu.async_copy(t1_h.at[ir_v.at[0, 0, pl.ds(0, tail)]],
                             dr_v.at[0, pl.ds(0, tail)], sg0).wait()
            pltpu.async_copy(t2_h.at[ic_v.at[0, 0, pl.ds(0, tail)]],
                             dr_v.at[0, pl.ds(0, tail)], sa0, add=True).wait()
            pltpu.sync_copy(dr_v.at[0, pl.ds(0, tail)],
                            gs_h.at[pl.ds(off, tail), pl.ds(0, 80)])

    return k(t1, t2, grow, gcol)


def _sc_scatter(svals, srow, n16):
    e = srow.shape[0]
    step = 2 * _CH
    epw = e // _NW
    nj = epw // step
    rem = epw - nj * step
    rpt = n16 // _NS
    mesh = plsc.VectorSubcoreMesh(core_axis_name="c", subcore_axis_name="s",
                                  num_cores=_NC, num_subcores=_NS)

    @functools.partial(
        pl.kernel,
        out_type=jax.ShapeDtypeStruct((_NC, n16, 80), jnp.float32),
        mesh=mesh,
        compiler_params=pltpu.CompilerParams(use_tc_tiling_on_sc=False),
        scratch_types=[
            pltpu.VMEM((2, 2, _CH), jnp.int32),
            pltpu.VMEM((1, max(rem, 8)), jnp.int32),
            pltpu.VMEM((2, step, 80), jnp.float32),
            pltpu.VMEM_SHARED((n16, 80), jnp.float32),
            pltpu.SemaphoreType.DMA,
            pltpu.SemaphoreType.DMA,
        ],
    )
    def k(s_h, srow_h, out_h, idx_v, idxt_v, val_v, acc_sh, sl0, sl1):
        c = lax.axis_index("c")
        s = lax.axis_index("s")
        wid = c * _NS + s
        base = wid * epw
        sls = (sl0, sl1)

        def zbody(i, carry):
            for kk in range(80 // _L):
                val_v[0, i, pl.ds(kk * _L, _L)] = jnp.zeros((_L,), jnp.float32)
            return carry

        lax.fori_loop(0, step, zbody, 0)
        nzfull = rpt // step
        zrem = rpt - nzfull * step
        for q in range(nzfull):
            pltpu.sync_copy(val_v.at[0],
                            acc_sh.at[pl.ds(s * rpt + q * step, step)])
        if zrem:
            pltpu.sync_copy(val_v.at[0, pl.ds(0, zrem)],
                            acc_sh.at[pl.ds(s * rpt + nzfull * step, zrem)])
        plsc.subcore_barrier()

        def fire_loads(slot, j):
            for g in range(2):
                pltpu.async_copy(srow_h.at[pl.ds(base + j * step + g * _CH, _CH)],
                                 idx_v.at[slot, g], sls[slot])
            pltpu.async_copy(s_h.at[pl.ds(base + j * step, step), pl.ds(0, 80)],
                             val_v.at[slot], sls[slot])

        def wait_loads(slot):
            for g in range(2):
                pltpu.make_async_copy(srow_h.at[pl.ds(base, _CH)],
                                      idx_v.at[slot, g], sls[slot]).wait()
            pltpu.make_async_copy(s_h.at[pl.ds(base, step), pl.ds(0, 80)],
                                  val_v.at[slot], sls[slot]).wait()

        def scatter(slot):
            for g in range(2):
                pltpu.sync_copy(val_v.at[slot, pl.ds(g * _CH, _CH)],
                                acc_sh.at[idx_v.at[slot, g]], add=True)

        fire_loads(0, 0)

        def body(jj, carry):
            a = jj * 2
            b = a + 1

            @pl.when(b < nj)
            def _():
                fire_loads(1, b)

            wait_loads(0)
            scatter(0)

            @pl.when(a + 2 < nj)
            def _():
                fire_loads(0, a + 2)

            @pl.when(b < nj)
            def _():
                wait_loads(1)
                scatter(1)

            return carry

        lax.fori_loop(0, (nj + 1) // 2, body, 0)

        if rem:
            toff = base + nj * step
            pltpu.sync_copy(srow_h.at[pl.ds(toff, rem)],
                            idxt_v.at[0, pl.ds(0, rem)])
            pltpu.sync_copy(s_h.at[pl.ds(toff, rem), pl.ds(0, 80)],
                            val_v.at[0, pl.ds(0, rem)])
            pltpu.sync_copy(val_v.at[0, pl.ds(0, rem)],
                            acc_sh.at[idxt_v.at[0]], add=True)

        plsc.subcore_barrier()
        pltpu.sync_copy(acc_sh.at[pl.ds(s * rpt, rpt)],
                        out_h.at[c].at[pl.ds(s * rpt, rpt)])

    return k(svals, srow)


def kernel(x, pos, edge_index, edge_attr, params):
    n, _ = x.shape
    h = params["w_in"].shape[1]
    n16 = n + 16

    row = edge_index[0]
    col = edge_index[1]
    cp = jnp.pad(pos, ((0, 0), (0, 13)))

    lp0 = params["layers"][0]
    hcur, t1, t2 = _tc_node_init(
        x, cp, params["w_in"], params["b_in"].reshape(1, h),
        lp0["we1"][0:h, :], lp0["we1"][h:2 * h, :])

    n_layers = len(params["layers"])
    for li, lp in enumerate(params["layers"]):
        gs = _sc_gather(t1, t2, row, col)
        svals = _tc_edge_mlp(
            gs, edge_attr,
            lp["we1"][2 * h:2 * h + 1, :],
            lp["we1"][2 * h + 1:, :],
            lp["be1"].reshape(1, h),
            lp["we2"], lp["be2"].reshape(1, h),
            lp["wc1"], lp["bc1"].reshape(1, h),
            lp["wc2"])
        acc2 = _sc_scatter(svals, row, n16)
        if li + 1 < n_layers:
            nxt = params["layers"][li + 1]
            w1a, w1b = nxt["we1"][0:h, :], nxt["we1"][h:2 * h, :]
        else:
            w1a, w1b = lp["we1"][0:h, :], lp["we1"][h:2 * h, :]
        hcur, t1, t2 = _tc_node_update(
            hcur, t1, acc2,
            lp["wn1"][0:h, :], lp["wn1"][h:2 * h, :],
            lp["bn1"].reshape(1, h),
            lp["wn2"], lp["bn2"].reshape(1, h),
            w1a, w1b)

    d_out = params["w_out"].shape[1]
    return _tc_final(hcur, params["w_out"], params["b_out"].reshape(1, d_out))


# speedup vs baseline: 1.0716x; 1.0716x over previous
import functools

import jax
import jax.numpy as jnp
from jax import lax
from jax.experimental import pallas as pl
from jax.experimental.pallas import tpu as pltpu
from jax.experimental.pallas import tpu_sc as plsc

_NC = 2
_NS = 16
_NW = _NC * _NS
_CH = 128
_GG = 2
_L = 16

_BN = 1000
_BE = 2048


def _silu(v):
    return v * jax.nn.sigmoid(v)


def _tc_node_init(x, cp, w_in, b_in, w1a, w1b):
    n, d_in = x.shape
    h = w_in.shape[1]
    grid = n // _BN

    def body(x_ref, cp_ref, win_ref, bin_ref, w1a_ref, w1b_ref,
             h_ref, t1_ref, t2_ref):
        h0 = jnp.dot(x_ref[...], win_ref[...],
                     preferred_element_type=jnp.float32) + bin_ref[...]
        h_ref[...] = h0
        cpv = cp_ref[...]
        p1 = jnp.dot(h0, w1a_ref[...], preferred_element_type=jnp.float32)
        p2 = jnp.dot(h0, w1b_ref[...], preferred_element_type=jnp.float32)
        t1_ref[...] = jnp.concatenate([p1, cpv], axis=1)
        t2_ref[...] = jnp.concatenate([p2, cpv], axis=1)

    return pl.pallas_call(
        body,
        grid=(grid,),
        in_specs=[
            pl.BlockSpec((_BN, d_in), lambda i: (i, 0)),
            pl.BlockSpec((_BN, 16), lambda i: (i, 0)),
            pl.BlockSpec((d_in, h), lambda i: (0, 0)),
            pl.BlockSpec((1, h), lambda i: (0, 0)),
            pl.BlockSpec((h, h), lambda i: (0, 0)),
            pl.BlockSpec((h, h), lambda i: (0, 0)),
        ],
        out_specs=[
            pl.BlockSpec((_BN, h), lambda i: (i, 0)),
            pl.BlockSpec((_BN, 80), lambda i: (i, 0)),
            pl.BlockSpec((_BN, 80), lambda i: (i, 0)),
        ],
        out_shape=[
            jax.ShapeDtypeStruct((n, h), jnp.float32),
            jax.ShapeDtypeStruct((n, 80), jnp.float32),
            jax.ShapeDtypeStruct((n, 80), jnp.float32),
        ],
    )(x, cp, w_in, b_in, w1a, w1b)


def _tc_edge_mlp(gs, ea, wr, w3, be1, we2, be2, wc1, bc1, wc2):
    e = gs.shape[0]
    h = 64
    d_edge = ea.shape[1]
    grid = (e + _BE - 1) // _BE

    def body(gs_ref, ea_ref, wr_ref, w3_ref, be1_ref,
             we2_ref, be2_ref, wc1_ref, bc1_ref, wc2_ref, s_ref):
        gsv = gs_ref[...]
        g12 = gsv[:, 0:64]
        diff = gsv[:, 64:80]
        lane = lax.broadcasted_iota(jnp.int32, (_BE, 16), 1)
        mask3 = (lane < 3).astype(jnp.float32)
        diffm = diff * mask3
        radial = jnp.sum(diffm * diffm, axis=1, keepdims=True)
        pre = (g12 + radial * wr_ref[...]
               + jnp.dot(ea_ref[...], w3_ref[...],
                         preferred_element_type=jnp.float32) + be1_ref[...])
        m1 = _silu(pre)
        m = _silu(jnp.dot(m1, we2_ref[...],
                          preferred_element_type=jnp.float32) + be2_ref[...])
        t1 = _silu(jnp.dot(m, wc1_ref[...],
                           preferred_element_type=jnp.float32) + bc1_ref[...])
        tt = jnp.dot(t1, wc2_ref[...], preferred_element_type=jnp.float32)
        right = diffm * tt + (lane == 3).astype(jnp.float32)
        z = jnp.zeros((_BE, 48), jnp.float32)
        s_ref[...] = jnp.concatenate([m, right, z], axis=1)

    return pl.pallas_call(
        body,
        grid=(grid,),
        in_specs=[
            pl.BlockSpec((_BE, 128), lambda i: (i, 0)),
            pl.BlockSpec((_BE, d_edge), lambda i: (i, 0)),
            pl.BlockSpec((1, h), lambda i: (0, 0)),
            pl.BlockSpec((d_edge, h), lambda i: (0, 0)),
            pl.BlockSpec((1, h), lambda i: (0, 0)),
            pl.BlockSpec((h, h), lambda i: (0, 0)),
            pl.BlockSpec((1, h), lambda i: (0, 0)),
            pl.BlockSpec((h, h), lambda i: (0, 0)),
            pl.BlockSpec((1, h), lambda i: (0, 0)),
            pl.BlockSpec((h, 1), lambda i: (0, 0)),
        ],
        out_specs=pl.BlockSpec((_BE, 128), lambda i: (i, 0)),
        out_shape=jax.ShapeDtypeStruct((e, 128), jnp.float32),
    )(gs, ea, wr, w3, be1, we2, be2, wc1, bc1, wc2)


def _tc_node_update(hcur, t1_prev, acc2, wn1a, wn1b, bn1, wn2, bn2, w1a, w1b):
    n, h = hcur.shape

    def body(h_ref, t1p_ref, a_ref, wn1a_ref, wn1b_ref, bn1_ref, wn2_ref,
             bn2_ref, w1a_ref, w1b_ref, ho_ref, t1_ref, t2_ref):
        acc = a_ref[0] + a_ref[1]
        m_agg = acc[:, 0:64]
        right = acc[:, 64:80]
        cnt = right[:, 3:4]
        lane = lax.broadcasted_iota(jnp.int32, (_BN, 16), 1)
        mask3 = (lane < 3).astype(jnp.float32)
        num = right * mask3
        cpn = t1p_ref[:, 64:80] + num / jnp.maximum(cnt, 1.0)
        pre = (jnp.dot(h_ref[...], wn1a_ref[...], preferred_element_type=jnp.float32)
               + jnp.dot(m_agg, wn1b_ref[...], preferred_element_type=jnp.float32)
               + bn1_ref[...])
        hn = jnp.dot(_silu(pre), wn2_ref[...],
                     preferred_element_type=jnp.float32) + bn2_ref[...]
        hnew = h_ref[...] + hn
        ho_ref[...] = hnew
        p1 = jnp.dot(hnew, w1a_ref[...], preferred_element_type=jnp.float32)
        p2 = jnp.dot(hnew, w1b_ref[...], preferred_element_type=jnp.float32)
        t1_ref[...] = jnp.concatenate([p1, cpn], axis=1)
        t2_ref[...] = jnp.concatenate([p2, cpn], axis=1)

    grid = n // _BN
    return pl.pallas_call(
        body,
        grid=(grid,),
        in_specs=[
            pl.BlockSpec((_BN, h), lambda i: (i, 0)),
            pl.BlockSpec((_BN, 80), lambda i: (i, 0)),
            pl.BlockSpec((2, _BN, 80), lambda i: (0, i, 0)),
            pl.BlockSpec((h, h), lambda i: (0, 0)),
            pl.BlockSpec((h, h), lambda i: (0, 0)),
            pl.BlockSpec((1, h), lambda i: (0, 0)),
            pl.BlockSpec((h, h), lambda i: (0, 0)),
            pl.BlockSpec((1, h), lambda i: (0, 0)),
            pl.BlockSpec((h, h), lambda i: (0, 0)),
            pl.BlockSpec((h, h), lambda i: (0, 0)),
        ],
        out_specs=[
            pl.BlockSpec((_BN, h), lambda i: (i, 0)),
            pl.BlockSpec((_BN, 80), lambda i: (i, 0)),
            pl.BlockSpec((_BN, 80), lambda i: (i, 0)),
        ],
        out_shape=[
            jax.ShapeDtypeStruct((n, h), jnp.float32),
            jax.ShapeDtypeStruct((n, 80), jnp.float32),
            jax.ShapeDtypeStruct((n, 80), jnp.float32),
        ],
    )(hcur, t1_prev, acc2, wn1a, wn1b, bn1, wn2, bn2, w1a, w1b)


def _tc_final(hcur, w_out, b_out):
    n, h = hcur.shape
    d_out = w_out.shape[1]
    grid = n // _BN

    def body(h_ref, w_ref, b_ref, o_ref):
        o_ref[...] = jnp.dot(h_ref[...], w_ref[...],
                             preferred_element_type=jnp.float32) + b_ref[...]

    return pl.pallas_call(
        body,
        grid=(grid,),
        in_specs=[
            pl.BlockSpec((_BN, h), lambda i: (i, 0)),
            pl.BlockSpec((h, d_out), lambda i: (0, 0)),
            pl.BlockSpec((1, d_out), lambda i: (0, 0)),
        ],
        out_specs=pl.BlockSpec((_BN, d_out), lambda i: (i, 0)),
        out_shape=jax.ShapeDtypeStruct((n, d_out), jnp.float32),
    )(hcur, w_out, b_out)


def _combine(dr_v, dc_v, slot, rows):
    def body(i, carry):
        for kk in range(80 // _L):
            sl = pl.ds(kk * _L, _L)
            a = dr_v[slot, i, sl]
            b = dc_v[slot, i, sl]
            if kk < 4:
                dr_v[slot, i, sl] = a + b
            else:
                dr_v[slot, i, sl] = a - b
        return carry

    lax.fori_loop(0, rows, body, 0)


def _sc_gather(t1, t2, grow, gcol):
    e = grow.shape[0]
    gg = 3
    step = gg * _CH
    epw = e // _NW
    nj = epw // step
    rem = epw - nj * step
    mesh = plsc.VectorSubcoreMesh(core_axis_name="c", subcore_axis_name="s",
                                  num_cores=_NC, num_subcores=_NS)

    @functools.partial(
        pl.kernel,
        out_type=jax.ShapeDtypeStruct((e, 128), jnp.float32),
        mesh=mesh,
        compiler_params=pltpu.CompilerParams(use_tc_tiling_on_sc=False),
        scratch_types=[
            pltpu.VMEM((2, gg, _CH), jnp.int32),
            pltpu.VMEM((2, gg, _CH), jnp.int32),
            pltpu.VMEM((2, step, 80), jnp.float32),
            pltpu.VMEM((2, step, 80), jnp.float32),
            pltpu.SemaphoreType.DMA,
            pltpu.SemaphoreType.DMA,
            pltpu.SemaphoreType.DMA,
            pltpu.SemaphoreType.DMA,
        ],
    )
    def k(t1_h, t2_h, grow_h, gcol_h, gs_h,
          ir_v, ic_v, dr_v, dc_v, sg0, sg1, sw0, sw1):
        c = lax.axis_index("c")
        s = lax.axis_index("s")
        wid = s * _NC + c
        base = wid * epw
        sgs = (sg0, sg1)
        sws = (sw0, sw1)

        def load_idx(slot, j):
            for g in range(gg):
                off = base + j * step + g * _CH
                pltpu.sync_copy(grow_h.at[pl.ds(off, _CH)], ir_v.at[slot, g])
                pltpu.sync_copy(gcol_h.at[pl.ds(off, _CH)], ic_v.at[slot, g])

        def fire(slot):
            for g in range(gg):
                pltpu.async_copy(t1_h.at[ir_v.at[slot, g]],
                                 dr_v.at[slot, pl.ds(g * _CH, _CH)], sgs[slot])
                pltpu.async_copy(t2_h.at[ic_v.at[slot, g]],
                                 dc_v.at[slot, pl.ds(g * _CH, _CH)], sgs[slot])

        def wait_g(slot):
            pltpu.make_async_copy(t1_h.at[ir_v.at[slot, 0]],
                                  dr_v.at[slot], sgs[slot]).wait()
            pltpu.make_async_copy(t2_h.at[ic_v.at[slot, 0]],
                                  dc_v.at[slot], sgs[slot]).wait()

        def write(slot, j):
            off = base + j * step
            pltpu.async_copy(dr_v.at[slot],
                             gs_h.at[pl.ds(off, step), pl.ds(0, 80)], sws[slot])

        def wait_w(slot):
            pltpu.make_async_copy(dr_v.at[slot],
                                  gs_h.at[pl.ds(base, step), pl.ds(0, 80)],
                                  sws[slot]).wait()

        load_idx(0, 0)
        fire(0)

        def body(jj, carry):
            a = jj * 2
            b = a + 1

            @pl.when(b < nj)
            def _():
                @pl.when(jj > 0)
                def _():
                    wait_w(1)

                load_idx(1, b)
                fire(1)

            wait_g(0)
            _combine(dr_v, dc_v, 0, step)
            write(0, a)

            @pl.when(a + 2 < nj)
            def _():
                wait_w(0)
                load_idx(0, a + 2)
                fire(0)

            @pl.when(b < nj)
            def _():
                wait_g(1)
                _combine(dr_v, dc_v, 1, step)
                write(1, b)

            return carry

        lax.fori_loop(0, (nj + 1) // 2, body, 0)
        wait_w(0)

        @pl.when(nj > 1)
        def _():
            wait_w(1)

        if rem:
            toff = base + nj * step
            pltpu.sync_copy(grow_h.at[pl.ds(toff, rem)], ir_v.at[0, 0, pl.ds(0, rem)])
            pltpu.sync_copy(gcol_h.at[pl.ds(toff, rem)], ic_v.at[0, 0, pl.ds(0, rem)])
            d1 = pltpu.async_copy(t1_h.at[ir_v.at[0, 0, pl.ds(0, rem)]],
                                  dr_v.at[0, pl.ds(0, rem)], sg0)
            d2 = pltpu.async_copy(t2_h.at[ic_v.at[0, 0, pl.ds(0, rem)]],
                                  dc_v.at[0, pl.ds(0, rem)], sg0)
            d1.wait()
            d2.wait()
            _combine(dr_v, dc_v, 0, rem)
            pltpu.sync_copy(dr_v.at[0, pl.ds(0, rem)],
                            gs_h.at[pl.ds(toff, rem), pl.ds(0, 80)])

    return k(t1, t2, grow, gcol)


def _sc_scatter(svals, srow, n16):
    e = srow.shape[0]
    step = _GG * _CH
    epw = e // _NW
    nj = epw // step
    rem = epw - nj * step
    rpt = n16 // _NS
    mesh = plsc.VectorSubcoreMesh(core_axis_name="c", subcore_axis_name="s",
                                  num_cores=_NC, num_subcores=_NS)

    @functools.partial(
        pl.kernel,
        out_type=jax.ShapeDtypeStruct((_NC, n16, 80), jnp.float32),
        mesh=mesh,
        compiler_params=pltpu.CompilerParams(use_tc_tiling_on_sc=False),
        scratch_types=[
            pltpu.VMEM((2, _GG, _CH), jnp.int32),
            pltpu.VMEM((1, max(rem, 8)), jnp.int32),
            pltpu.VMEM((2, step, 80), jnp.float32),
            pltpu.VMEM_SHARED((n16, 80), jnp.float32),
            pltpu.SemaphoreType.DMA,
            pltpu.SemaphoreType.DMA,
        ],
    )
    def k(s_h, srow_h, out_h, idx_v, idxt_v, val_v, acc_sh, sl0, sl1):
        c = lax.axis_index("c")
        s = lax.axis_index("s")
        wid = c * _NS + s
        base = wid * epw
        sls = (sl0, sl1)

        def zbody(i, carry):
            for kk in range(80 // _L):
                val_v[0, i, pl.ds(kk * _L, _L)] = jnp.zeros((_L,), jnp.float32)
            return carry

        lax.fori_loop(0, step, zbody, 0)
        nzfull = rpt // step
        zrem = rpt - nzfull * step
        for q in range(nzfull):
            pltpu.sync_copy(val_v.at[0],
                            acc_sh.at[pl.ds(s * rpt + q * step, step)])
        if zrem:
            pltpu.sync_copy(val_v.at[0, pl.ds(0, zrem)],
                            acc_sh.at[pl.ds(s * rpt + nzfull * step, zrem)])
        plsc.subcore_barrier()

        def fire_loads(slot, j):
            for g in range(_GG):
                pltpu.async_copy(srow_h.at[pl.ds(base + j * step + g * _CH, _CH)],
                                 idx_v.at[slot, g], sls[slot])
            pltpu.async_copy(s_h.at[pl.ds(base + j * step, step), pl.ds(0, 80)],
                             val_v.at[slot], sls[slot])

        def wait_loads(slot):
            for g in range(_GG):
                pltpu.make_async_copy(srow_h.at[pl.ds(base, _CH)],
                                      idx_v.at[slot, g], sls[slot]).wait()
            pltpu.make_async_copy(s_h.at[pl.ds(base, step), pl.ds(0, 80)],
                                  val_v.at[slot], sls[slot]).wait()

        def scatter(slot):
            for g in range(_GG):
                pltpu.sync_copy(val_v.at[slot, pl.ds(g * _CH, _CH)],
                                acc_sh.at[idx_v.at[slot, g]], add=True)

        fire_loads(0, 0)

        def body(jj, carry):
            a = jj * 2
            b = a + 1

            @pl.when(b < nj)
            def _():
                fire_loads(1, b)

            wait_loads(0)
            scatter(0)

            @pl.when(a + 2 < nj)
            def _():
                fire_loads(0, a + 2)

            @pl.when(b < nj)
            def _():
                wait_loads(1)
                scatter(1)

            return carry

        lax.fori_loop(0, (nj + 1) // 2, body, 0)

        if rem:
            toff = base + nj * step
            pltpu.sync_copy(srow_h.at[pl.ds(toff, rem)],
                            idxt_v.at[0, pl.ds(0, rem)])
            pltpu.sync_copy(s_h.at[pl.ds(toff, rem), pl.ds(0, 80)],
                            val_v.at[0, pl.ds(0, rem)])
            pltpu.sync_copy(val_v.at[0, pl.ds(0, rem)],
                            acc_sh.at[idxt_v.at[0]], add=True)

        plsc.subcore_barrier()
        pltpu.sync_copy(acc_sh.at[pl.ds(s * rpt, rpt)],
                        out_h.at[c].at[pl.ds(s * rpt, rpt)])

    return k(svals, srow)


def kernel(x, pos, edge_index, edge_attr, params):
    n, _ = x.shape
    h = params["w_in"].shape[1]
    n16 = n + 16

    row = edge_index[0]
    col = edge_index[1]
    cp = jnp.pad(pos, ((0, 0), (0, 13)))

    lp0 = params["layers"][0]
    hcur, t1, t2 = _tc_node_init(
        x, cp, params["w_in"], params["b_in"].reshape(1, h),
        lp0["we1"][0:h, :], lp0["we1"][h:2 * h, :])

    n_layers = len(params["layers"])
    for li, lp in enumerate(params["layers"]):
        gs = _sc_gather(t1, t2, row, col)
        svals = _tc_edge_mlp(
            gs, edge_attr,
            lp["we1"][2 * h:2 * h + 1, :],
            lp["we1"][2 * h + 1:, :],
            lp["be1"].reshape(1, h),
            lp["we2"], lp["be2"].reshape(1, h),
            lp["wc1"], lp["bc1"].reshape(1, h),
            lp["wc2"])
        acc2 = _sc_scatter(svals, row, n16)
        if li + 1 < n_layers:
            nxt = params["layers"][li + 1]
            w1a, w1b = nxt["we1"][0:h, :], nxt["we1"][h:2 * h, :]
        else:
            w1a, w1b = lp["we1"][0:h, :], lp["we1"][h:2 * h, :]
        hcur, t1, t2 = _tc_node_update(
            hcur, t1, acc2,
            lp["wn1"][0:h, :], lp["wn1"][h:2 * h, :],
            lp["bn1"].reshape(1, h),
            lp["wn2"], lp["bn2"].reshape(1, h),
            w1a, w1b)

    d_out = params["w_out"].shape[1]
    return _tc_final(hcur, params["w_out"], params["b_out"].reshape(1, d_out))
